# baseline (speedup 1.0000x reference)
import functools

import jax
import jax.numpy as jnp
import numpy as np
from jax import lax
from jax.experimental import pallas as pl
from jax.experimental.pallas import tpu as pltpu
from jax.experimental.pallas import tpu_sc as plsc

KS = 3
PAD = 1
C = 96
NTAP = 9
H = 222
HW = H * H
HP = 226
NROWS = HP * HP
RPAD = 51200
NFLAT = 224 * 224

NW = 32
P_TILE = 1584
NPIX = NW * P_TILE
PB = 8
NB = P_TILE // PB
TAPS4 = 4 * NTAP
BLK = PB * TAPS4
GCH = 128
NG = (BLK + GCH - 1) // GCH
NPIX2 = NPIX + 4 * PB

_mesh = plsc.VectorSubcoreMesh(core_axis_name="c", subcore_axis_name="s")


@functools.partial(
    pl.kernel,
    mesh=_mesh,
    out_type=jax.ShapeDtypeStruct((NPIX, C), jnp.float32),
    scratch_types=[
        pltpu.VMEM((BLK,), jnp.int32),
        pltpu.VMEM((BLK,), jnp.int32),
        pltpu.VMEM((BLK,), jnp.int32),
        pltpu.VMEM((BLK + 16,), jnp.float32),
        pltpu.VMEM((BLK + 16,), jnp.float32),
        pltpu.VMEM((BLK + 16,), jnp.float32),
        pltpu.VMEM((BLK, 128), jnp.float32),
        pltpu.VMEM((BLK, 128), jnp.float32),
        pltpu.VMEM((BLK, 128), jnp.float32),
        pltpu.VMEM((PB, C), jnp.float32),
        pltpu.SemaphoreType.DMA,
        pltpu.SemaphoreType.DMA,
        pltpu.SemaphoreType.DMA,
        pltpu.SemaphoreType.DMA,
        pltpu.SemaphoreType.DMA,
        pltpu.SemaphoreType.DMA,
    ],
)
def _sc_gather_combine(table_hbm, idx_hbm, w_hbm, out_hbm,
                       idx_v0, idx_v1, idx_v2, w_v0, w_v1, w_v2,
                       rows_v0, rows_v1, rows_v2, out_v,
                       sem_l0, sem_l1, sem_l2, sem_g0, sem_g1, sem_g2):
    wid = lax.axis_index("s") * 2 + lax.axis_index("c")
    base_pix = wid * P_TILE
    idx_v = (idx_v0, idx_v1, idx_v2)
    w_v = (w_v0, w_v1, w_v2)
    rows_v = (rows_v0, rows_v1, rows_v2)
    sem_l = (sem_l0, sem_l1, sem_l2)
    sem_g = (sem_g0, sem_g1, sem_g2)

    def load_copies(b, buf):
        off = (base_pix + b * PB) * TAPS4
        return (pltpu.make_async_copy(idx_hbm.at[pl.ds(off, BLK)],
                                      idx_v[buf], sem_l[buf]),
                pltpu.make_async_copy(w_hbm.at[pl.ds(off, BLK)],
                                     w_v[buf].at[pl.ds(0, BLK)], sem_l[buf]))

    def gather_copies(buf):
        out = []
        for k in range(NG):
            cnt = min(GCH, BLK - k * GCH)
            out.append(pltpu.make_async_copy(
                table_hbm.at[idx_v[buf].at[pl.ds(k * GCH, cnt)]],
                rows_v[buf].at[pl.ds(k * GCH, cnt)], sem_g[buf]))
        return out

    for blk in range(3):
        for c0 in load_copies(blk, blk):
            c0.start()
    for blk in range(2):
        for c0 in load_copies(blk, blk):
            c0.wait()
        for c0 in gather_copies(blk):
            c0.start()

    def combine(buf, b):
        def pix_body(i, c2):
            base = i * TAPS4
            wvecs = tuple(w_v[buf][pl.ds(base + 16 * g, 16)] for g in range(3))
            acc = [jnp.zeros((16,), jnp.float32) for _ in range(C // 16)]
            for t in range(TAPS4):
                wv = lax.broadcast_in_dim(wvecs[t // 16][t % 16], (16,), ())
                for c in range(C // 16):
                    acc[c] = acc[c] + wv * rows_v[buf][base + t,
                                                       pl.ds(c * 16, 16)]
            for c in range(C // 16):
                out_v[i, pl.ds(c * 16, 16)] = acc[c]
            return c2
        lax.fori_loop(0, PB, pix_body, 0)
        pltpu.sync_copy(out_v, out_hbm.at[pl.ds(base_pix + b * PB, PB)])

    def triple(i, carry):
        for phase in range(3):
            b = 3 * i + phase
            buf2 = (phase + 2) % 3
            for c0 in load_copies(b + 2, buf2):
                c0.wait()
            for c0 in gather_copies(buf2):
                c0.start()
            for c0 in gather_copies(phase):
                c0.wait()
            combine(phase, b)
            for c0 in load_copies(b + 3, phase):
                c0.start()
        return carry

    lax.fori_loop(0, NB // 3, triple, 0)

    for blk in (NB, NB + 1):
        for c0 in gather_copies(blk % 3):
            c0.wait()
    for c0 in load_copies(NB + 2, (NB + 2) % 3):
        c0.wait()


def _offset_conv_body(x9_ref, w_ref, b_ref, o_ref):
    o_ref[...] = jnp.dot(w_ref[...], x9_ref[...],
                         preferred_element_type=jnp.float32) + b_ref[...]


def _offset_conv(x9, W9, b9):
    return pl.pallas_call(
        _offset_conv_body,
        grid=(NFLAT // 512,),
        in_specs=[pl.BlockSpec((9 * C, 512), lambda j: (0, j)),
                  pl.BlockSpec((32, 9 * C), lambda j: (0, 0)),
                  pl.BlockSpec((32, 1), lambda j: (0, 0))],
        out_specs=pl.BlockSpec((32, 512), lambda j: (0, j)),
        out_shape=jax.ShapeDtypeStruct((32, NFLAT), jnp.float32),
    )(x9, W9, b9)


def _table_body(x_ref, w_ref, o_ref):
    o_ref[0] = jnp.dot(x_ref[...], w_ref[0],
                       preferred_element_type=jnp.float32)


def _table_matmul(x_padT, Wn):
    return pl.pallas_call(
        _table_body,
        grid=(RPAD // 512, NTAP),
        in_specs=[pl.BlockSpec((512, C), lambda r, n: (r, 0)),
                  pl.BlockSpec((1, C, 128), lambda r, n: (n, 0, 0))],
        out_specs=pl.BlockSpec((1, 512, 128), lambda r, n: (n, r, 0)),
        out_shape=jax.ShapeDtypeStruct((NTAP, RPAD, 128), jnp.float32),
    )(x_padT, Wn)


HWP = 49664


def _iw_body(ox_ref, oy_ref, dd_ref, hh_ref, ww_ref, idx_ref, wts_ref):
    dd = dd_ref[...]
    px = ox_ref[...] + dd[:, 0:1] + hh_ref[...]
    py = oy_ref[...] + dd[:, 1:2] + ww_ref[...]

    def axis_terms(p):
        f = jnp.floor(p)
        q0 = jnp.clip(f, 0.0, HP - 1.0)
        q1 = jnp.clip(f + 1.0, 0.0, HP - 1.0)
        masked = jnp.logical_or(p < float(PAD), p > HP - 1.0 - PAD)
        pu = jnp.clip(jnp.where(masked, f, p), 0.0, HP - 1.0)
        return q0, q1, 1.0 + q0 - pu, 1.0 - (q1 - pu)

    qx0, qx1, wx0, wx1 = axis_terms(px)
    qy0, qy1, wy0, wy1 = axis_terms(py)
    nbase = dd[:, 2:3]
    for t, (qx, qy, wx, wy) in enumerate(
            [(qx0, qy0, wx0, wy0), (qx1, qy1, wx1, wy1),
             (qx0, qy1, wx0, wy1), (qx1, qy0, wx1, wy0)]):
        idx_ref[t] = (nbase + qx * float(HP) + qy).astype(jnp.int32)
        wts_ref[t] = wx * wy


def _iw_kernel(ox, oy, dd, hh, ww):
    return pl.pallas_call(
        _iw_body,
        grid=(HWP // 512,),
        in_specs=[pl.BlockSpec((NTAP, 512), lambda j: (0, j)),
                  pl.BlockSpec((NTAP, 512), lambda j: (0, j)),
                  pl.BlockSpec((NTAP, 4), lambda j: (0, 0)),
                  pl.BlockSpec((1, 512), lambda j: (0, j)),
                  pl.BlockSpec((1, 512), lambda j: (0, j))],
        out_specs=[pl.BlockSpec((4, NTAP, 512), lambda j: (0, 0, j)),
                   pl.BlockSpec((4, NTAP, 512), lambda j: (0, 0, j))],
        out_shape=[jax.ShapeDtypeStruct((4, NTAP, HWP), jnp.int32),
                   jax.ShapeDtypeStruct((4, NTAP, HWP), jnp.float32)],
    )(ox, oy, dd, hh, ww)


def _indices_and_weights(offset):
    ox = jnp.pad(offset[0::2].reshape(NTAP, HW), ((0, 0), (0, HWP - HW)))
    oy = jnp.pad(offset[1::2].reshape(NTAP, HW), ((0, 0), (0, HWP - HW)))
    j = np.arange(HWP)
    hh = jnp.asarray((np.minimum(j // H, H - 1) + 1).reshape(1, -1),
                     jnp.float32)
    ww = jnp.asarray((j % H + 1).reshape(1, -1), jnp.float32)
    dd = np.zeros((NTAP, 4), np.float32)
    dd[:, 0] = np.repeat(np.arange(-1, 2), 3)
    dd[:, 1] = np.tile(np.arange(-1, 2), 3)
    dd[:, 2] = np.arange(NTAP) * float(RPAD)
    idx4, wts4 = _iw_kernel(ox, oy, jnp.asarray(dd), hh, ww)
    idx = jnp.transpose(idx4[:, :, :HW], (2, 1, 0)).reshape(HW, TAPS4)
    wts = jnp.transpose(wts4[:, :, :HW], (2, 1, 0)).reshape(HW, TAPS4)
    return idx, wts


def kernel(x, W_off, b_off, W_d, b_d):
    x_flat = x[0].reshape(C, NFLAT)
    x_flat_p = jnp.pad(x_flat, ((0, 0), (0, 512)))
    shifts = [i * 224 + j for i in range(3) for j in range(3)]
    x9 = jnp.concatenate([x_flat_p[:, s:s + NFLAT] for s in shifts], axis=0)
    W9 = jnp.pad(jnp.transpose(W_off, (0, 2, 3, 1)).reshape(18, 9 * C),
                 ((0, 14), (0, 0)))
    b9 = jnp.pad(b_off, (0, 14)).reshape(32, 1)
    off_flat = _offset_conv(x9, W9, b9)
    offset = off_flat[:18].reshape(18, 224, 224)[:, :222, :222]

    idx, wts = _indices_and_weights(offset)
    idx = jnp.pad(idx, ((0, NPIX2 - HW), (0, 0))).reshape(-1)
    wts = jnp.pad(wts, ((0, NPIX2 - HW), (0, 0))).reshape(-1)

    x_pad = jnp.pad(x[0], ((0, 0), (PAD, PAD), (PAD, PAD)))
    x_padT = jnp.pad(x_pad.reshape(C, NROWS).T, ((0, RPAD - NROWS), (0, 0)))
    Wn = jnp.transpose(W_d.reshape(C, C, NTAP), (2, 1, 0))
    Wn = jnp.pad(Wn, ((0, 0), (0, 0), (0, 128 - C)))
    table = _table_matmul(x_padT, Wn).reshape(NTAP * RPAD, 128)

    out_rows = _sc_gather_combine(table, idx, wts)

    out = out_rows[:HW].T + b_d[:, None]
    return out.reshape(1, C, H, H)


# speedup vs baseline: 1.0135x; 1.0135x over previous
import functools

import jax
import jax.numpy as jnp
import numpy as np
from jax import lax
from jax.experimental import pallas as pl
from jax.experimental.pallas import tpu as pltpu
from jax.experimental.pallas import tpu_sc as plsc

KS = 3
PAD = 1
C = 96
NTAP = 9
H = 222
HW = H * H
HP = 226
NROWS = HP * HP
RPAD = 51200
NFLAT = 224 * 224

NW = 32
P_TILE = 1584
NPIX = NW * P_TILE
PB = 24
NB = P_TILE // PB
TAPS4 = 4 * NTAP
BLK = PB * TAPS4
GCH = 128
NG = (BLK + GCH - 1) // GCH
NPIX2 = NPIX + 4 * PB

_mesh = plsc.VectorSubcoreMesh(core_axis_name="c", subcore_axis_name="s")


@functools.partial(
    pl.kernel,
    mesh=_mesh,
    out_type=jax.ShapeDtypeStruct((NPIX, C), jnp.float32),
    scratch_types=[
        pltpu.VMEM((BLK,), jnp.int32),
        pltpu.VMEM((BLK + 16,), jnp.float32),
        pltpu.VMEM((BLK, 128), jnp.float32),
        pltpu.VMEM((PB, C), jnp.float32),
        pltpu.SemaphoreType.DMA,
    ],
)
def _sc_gather_combine(table_hbm, idx_hbm, w_hbm, out_hbm,
                       idx_v, w_v, rows_v, out_v, sem):
    wid = lax.axis_index("s") * 2 + lax.axis_index("c")
    base_pix = wid * P_TILE

    def block(b, carry):
        off = (base_pix + b * PB) * TAPS4
        pltpu.sync_copy(idx_hbm.at[pl.ds(off, BLK)], idx_v)
        pltpu.sync_copy(w_hbm.at[pl.ds(off, BLK)], w_v.at[pl.ds(0, BLK)])
        handles = []
        for k in range(NG):
            cnt = min(GCH, BLK - k * GCH)
            handles.append(pltpu.async_copy(
                table_hbm.at[idx_v.at[pl.ds(k * GCH, cnt)]],
                rows_v.at[pl.ds(k * GCH, cnt)], sem))
        for h in handles:
            h.wait()

        def pix_body(i, c2):
            base = i * TAPS4
            wvecs = tuple(w_v[pl.ds(base + 16 * g, 16)] for g in range(3))
            acc = [jnp.zeros((16,), jnp.float32) for _ in range(C // 16)]
            for t in range(TAPS4):
                wv = lax.broadcast_in_dim(wvecs[t // 16][t % 16], (16,), ())
                for c in range(C // 16):
                    acc[c] = acc[c] + wv * rows_v[base + t, pl.ds(c * 16, 16)]
            for c in range(C // 16):
                out_v[i, pl.ds(c * 16, 16)] = acc[c]
            return c2
        lax.fori_loop(0, PB, pix_body, 0)
        pltpu.sync_copy(out_v, out_hbm.at[pl.ds(base_pix + b * PB, PB)])
        return carry

    lax.fori_loop(0, NB, block, 0)


def _offset_conv_body(x9_ref, w_ref, b_ref, o_ref):
    o_ref[...] = jnp.dot(w_ref[...], x9_ref[...],
                         preferred_element_type=jnp.float32) + b_ref[...]


def _offset_conv(x9, W9, b9):
    return pl.pallas_call(
        _offset_conv_body,
        grid=(NFLAT // 512,),
        in_specs=[pl.BlockSpec((9 * C, 512), lambda j: (0, j)),
                  pl.BlockSpec((32, 9 * C), lambda j: (0, 0)),
                  pl.BlockSpec((32, 1), lambda j: (0, 0))],
        out_specs=pl.BlockSpec((32, 512), lambda j: (0, j)),
        out_shape=jax.ShapeDtypeStruct((32, NFLAT), jnp.float32),
    )(x9, W9, b9)


def _table_body(x_ref, w_ref, o_ref):
    o_ref[0] = jnp.dot(x_ref[...], w_ref[0],
                       preferred_element_type=jnp.float32)


def _table_matmul(x_padT, Wn):
    return pl.pallas_call(
        _table_body,
        grid=(RPAD // 512, NTAP),
        in_specs=[pl.BlockSpec((512, C), lambda r, n: (r, 0)),
                  pl.BlockSpec((1, C, 128), lambda r, n: (n, 0, 0))],
        out_specs=pl.BlockSpec((1, 512, 128), lambda r, n: (n, r, 0)),
        out_shape=jax.ShapeDtypeStruct((NTAP, RPAD, 128), jnp.float32),
    )(x_padT, Wn)


HWP = 49664


def _iw_body(ox_ref, oy_ref, dd_ref, hh_ref, ww_ref, idx_ref, wts_ref):
    dd = dd_ref[...]
    px = ox_ref[...] + dd[:, 0:1] + hh_ref[...]
    py = oy_ref[...] + dd[:, 1:2] + ww_ref[...]

    def axis_terms(p):
        f = jnp.floor(p)
        q0 = jnp.clip(f, 0.0, HP - 1.0)
        q1 = jnp.clip(f + 1.0, 0.0, HP - 1.0)
        masked = jnp.logical_or(p < float(PAD), p > HP - 1.0 - PAD)
        pu = jnp.clip(jnp.where(masked, f, p), 0.0, HP - 1.0)
        return q0, q1, 1.0 + q0 - pu, 1.0 - (q1 - pu)

    qx0, qx1, wx0, wx1 = axis_terms(px)
    qy0, qy1, wy0, wy1 = axis_terms(py)
    nbase = dd[:, 2:3]
    for t, (qx, qy, wx, wy) in enumerate(
            [(qx0, qy0, wx0, wy0), (qx1, qy1, wx1, wy1),
             (qx0, qy1, wx0, wy1), (qx1, qy0, wx1, wy0)]):
        idx_ref[t] = (nbase + qx * float(HP) + qy).astype(jnp.int32)
        wts_ref[t] = wx * wy


def _iw_kernel(ox, oy, dd, hh, ww):
    return pl.pallas_call(
        _iw_body,
        grid=(HWP // 512,),
        in_specs=[pl.BlockSpec((NTAP, 512), lambda j: (0, j)),
                  pl.BlockSpec((NTAP, 512), lambda j: (0, j)),
                  pl.BlockSpec((NTAP, 4), lambda j: (0, 0)),
                  pl.BlockSpec((1, 512), lambda j: (0, j)),
                  pl.BlockSpec((1, 512), lambda j: (0, j))],
        out_specs=[pl.BlockSpec((4, NTAP, 512), lambda j: (0, 0, j)),
                   pl.BlockSpec((4, NTAP, 512), lambda j: (0, 0, j))],
        out_shape=[jax.ShapeDtypeStruct((4, NTAP, HWP), jnp.int32),
                   jax.ShapeDtypeStruct((4, NTAP, HWP), jnp.float32)],
    )(ox, oy, dd, hh, ww)


def _indices_and_weights(offset):
    ox = jnp.pad(offset[0::2].reshape(NTAP, HW), ((0, 0), (0, HWP - HW)))
    oy = jnp.pad(offset[1::2].reshape(NTAP, HW), ((0, 0), (0, HWP - HW)))
    j = np.arange(HWP)
    hh = jnp.asarray((np.minimum(j // H, H - 1) + 1).reshape(1, -1),
                     jnp.float32)
    ww = jnp.asarray((j % H + 1).reshape(1, -1), jnp.float32)
    dd = np.zeros((NTAP, 4), np.float32)
    dd[:, 0] = np.repeat(np.arange(-1, 2), 3)
    dd[:, 1] = np.tile(np.arange(-1, 2), 3)
    dd[:, 2] = np.arange(NTAP) * float(RPAD)
    idx4, wts4 = _iw_kernel(ox, oy, jnp.asarray(dd), hh, ww)
    idx = jnp.transpose(idx4[:, :, :HW], (2, 1, 0)).reshape(HW, TAPS4)
    wts = jnp.transpose(wts4[:, :, :HW], (2, 1, 0)).reshape(HW, TAPS4)
    return idx, wts


def kernel(x, W_off, b_off, W_d, b_d):
    x_flat = x[0].reshape(C, NFLAT)
    x_flat_p = jnp.pad(x_flat, ((0, 0), (0, 512)))
    shifts = [i * 224 + j for i in range(3) for j in range(3)]
    x9 = jnp.concatenate([x_flat_p[:, s:s + NFLAT] for s in shifts], axis=0)
    W9 = jnp.pad(jnp.transpose(W_off, (0, 2, 3, 1)).reshape(18, 9 * C),
                 ((0, 14), (0, 0)))
    b9 = jnp.pad(b_off, (0, 14)).reshape(32, 1)
    off_flat = _offset_conv(x9, W9, b9)
    offset = off_flat[:18].reshape(18, 224, 224)[:, :222, :222]

    idx, wts = _indices_and_weights(offset)
    idx = jnp.pad(idx, ((0, NPIX2 - HW), (0, 0))).reshape(-1)
    wts = jnp.pad(wts, ((0, NPIX2 - HW), (0, 0))).reshape(-1)

    x_pad = jnp.pad(x[0], ((0, 0), (PAD, PAD), (PAD, PAD)))
    x_padT = jnp.pad(x_pad.reshape(C, NROWS).T, ((0, RPAD - NROWS), (0, 0)))
    Wn = jnp.transpose(W_d.reshape(C, C, NTAP), (2, 1, 0))
    Wn = jnp.pad(Wn, ((0, 0), (0, 0), (0, 128 - C)))
    table = _table_matmul(x_padT, Wn).reshape(NTAP * RPAD, 128)

    out_rows = _sc_gather_combine(table, idx, wts)

    out = out_rows[:HW].T + b_d[:, None]
    return out.reshape(1, C, H, H)


# speedup vs baseline: 1.2555x; 1.2387x over previous
import functools

import jax
import jax.numpy as jnp
import numpy as np
from jax import lax
from jax.experimental import pallas as pl
from jax.experimental.pallas import tpu as pltpu
from jax.experimental.pallas import tpu_sc as plsc

KS = 3
PAD = 1
C = 96
NTAP = 9
H = 222
HW = H * H
HP = 226
NROWS = HP * HP
RPAD = 51200
NFLAT = 224 * 224

NW = 32
P_TILE = 1568
NPIX = NW * P_TILE
PB = 16
NB = P_TILE // PB
TAPS4 = 4 * NTAP
BLK = PB * TAPS4
GCH = 128
NG = (BLK + GCH - 1) // GCH
NPIX2 = NPIX + 4 * PB

_mesh = plsc.VectorSubcoreMesh(core_axis_name="c", subcore_axis_name="s")


@functools.partial(
    pl.kernel,
    mesh=_mesh,
    out_type=jax.ShapeDtypeStruct((NPIX, C), jnp.float32),
    scratch_types=[
        pltpu.VMEM((BLK,), jnp.int32),
        pltpu.VMEM((BLK + 16,), jnp.float32),
        pltpu.VMEM((BLK, 128), jnp.float32),
        pltpu.VMEM((PB, C), jnp.float32),
        pltpu.SemaphoreType.DMA,
    ],
)
def _sc_gather_combine(table_hbm, idx_hbm, w_hbm, out_hbm,
                       idx_v, w_v, rows_v, out_v, sem):
    wid = lax.axis_index("s") * 2 + lax.axis_index("c")
    base_pix = wid * P_TILE

    def block(b, carry):
        off = (base_pix + b * PB) * TAPS4
        pltpu.sync_copy(idx_hbm.at[pl.ds(off, BLK)], idx_v)
        pltpu.sync_copy(w_hbm.at[pl.ds(off, BLK)], w_v.at[pl.ds(0, BLK)])
        handles = []
        for k in range(NG):
            cnt = min(GCH, BLK - k * GCH)
            handles.append(pltpu.async_copy(
                table_hbm.at[idx_v.at[pl.ds(k * GCH, cnt)]],
                rows_v.at[pl.ds(k * GCH, cnt)], sem))
        for h in handles:
            h.wait()

        def pix_body(i, c2):
            base = i * TAPS4
            wvecs = tuple(w_v[pl.ds(base + 16 * g, 16)] for g in range(3))
            acc = [jnp.zeros((16,), jnp.float32) for _ in range(C // 16)]
            for t in range(TAPS4):
                wv = lax.broadcast_in_dim(wvecs[t // 16][t % 16], (16,), ())
                for c in range(C // 16):
                    acc[c] = acc[c] + wv * rows_v[base + t, pl.ds(c * 16, 16)]
            for c in range(C // 16):
                out_v[i, pl.ds(c * 16, 16)] = acc[c]
            return c2
        lax.fori_loop(0, PB, pix_body, 0)
        pltpu.sync_copy(out_v, out_hbm.at[pl.ds(base_pix + b * PB, PB)])
        return carry

    lax.fori_loop(0, NB, block, 0)


def _offset_conv_body(x9_ref, w_ref, b_ref, o_ref):
    o_ref[...] = jnp.dot(w_ref[...], x9_ref[...],
                         preferred_element_type=jnp.float32) + b_ref[...]


def _offset_conv(x9, W9, b9):
    return pl.pallas_call(
        _offset_conv_body,
        grid=(NFLAT // 512,),
        in_specs=[pl.BlockSpec((9 * C, 512), lambda j: (0, j)),
                  pl.BlockSpec((32, 9 * C), lambda j: (0, 0)),
                  pl.BlockSpec((32, 1), lambda j: (0, 0))],
        out_specs=pl.BlockSpec((32, 512), lambda j: (0, j)),
        out_shape=jax.ShapeDtypeStruct((32, NFLAT), jnp.float32),
    )(x9, W9, b9)


def _table_body(x_ref, w_ref, o_ref):
    o_ref[0] = jnp.dot(x_ref[...], w_ref[0],
                       preferred_element_type=jnp.float32)


def _table_matmul(x_padT, Wn):
    return pl.pallas_call(
        _table_body,
        grid=(RPAD // 512, NTAP),
        in_specs=[pl.BlockSpec((512, C), lambda r, n: (r, 0)),
                  pl.BlockSpec((1, C, 128), lambda r, n: (n, 0, 0))],
        out_specs=pl.BlockSpec((1, 512, 128), lambda r, n: (n, r, 0)),
        out_shape=jax.ShapeDtypeStruct((NTAP, RPAD, 128), jnp.float32),
    )(x_padT, Wn)


HWP = 49664


def _iw_body(ox_ref, oy_ref, dd_ref, hh_ref, ww_ref, idx_ref, wts_ref):
    dd = dd_ref[...]
    px = ox_ref[...] + dd[:, 0:1] + hh_ref[...]
    py = oy_ref[...] + dd[:, 1:2] + ww_ref[...]

    def axis_terms(p):
        f = jnp.floor(p)
        q0 = jnp.clip(f, 0.0, HP - 1.0)
        q1 = jnp.clip(f + 1.0, 0.0, HP - 1.0)
        masked = jnp.logical_or(p < float(PAD), p > HP - 1.0 - PAD)
        pu = jnp.clip(jnp.where(masked, f, p), 0.0, HP - 1.0)
        return q0, q1, 1.0 + q0 - pu, 1.0 - (q1 - pu)

    qx0, qx1, wx0, wx1 = axis_terms(px)
    qy0, qy1, wy0, wy1 = axis_terms(py)
    nbase = dd[:, 2:3]
    for t, (qx, qy, wx, wy) in enumerate(
            [(qx0, qy0, wx0, wy0), (qx1, qy1, wx1, wy1),
             (qx0, qy1, wx0, wy1), (qx1, qy0, wx1, wy0)]):
        idx_ref[t] = (nbase + qx * float(HP) + qy).astype(jnp.int32)
        wts_ref[t] = wx * wy


def _iw_kernel(ox, oy, dd, hh, ww):
    return pl.pallas_call(
        _iw_body,
        grid=(HWP // 512,),
        in_specs=[pl.BlockSpec((NTAP, 512), lambda j: (0, j)),
                  pl.BlockSpec((NTAP, 512), lambda j: (0, j)),
                  pl.BlockSpec((NTAP, 4), lambda j: (0, 0)),
                  pl.BlockSpec((1, 512), lambda j: (0, j)),
                  pl.BlockSpec((1, 512), lambda j: (0, j))],
        out_specs=[pl.BlockSpec((4, NTAP, 512), lambda j: (0, 0, j)),
                   pl.BlockSpec((4, NTAP, 512), lambda j: (0, 0, j))],
        out_shape=[jax.ShapeDtypeStruct((4, NTAP, HWP), jnp.int32),
                   jax.ShapeDtypeStruct((4, NTAP, HWP), jnp.float32)],
    )(ox, oy, dd, hh, ww)


def _indices_and_weights(offset):
    ox = jnp.pad(offset[0::2].reshape(NTAP, HW), ((0, 0), (0, HWP - HW)))
    oy = jnp.pad(offset[1::2].reshape(NTAP, HW), ((0, 0), (0, HWP - HW)))
    j = np.arange(HWP)
    hh = jnp.asarray((np.minimum(j // H, H - 1) + 1).reshape(1, -1),
                     jnp.float32)
    ww = jnp.asarray((j % H + 1).reshape(1, -1), jnp.float32)
    dd = np.zeros((NTAP, 4), np.float32)
    dd[:, 0] = np.repeat(np.arange(-1, 2), 3)
    dd[:, 1] = np.tile(np.arange(-1, 2), 3)
    dd[:, 2] = np.arange(NTAP) * float(RPAD)
    idx4, wts4 = _iw_kernel(ox, oy, jnp.asarray(dd), hh, ww)
    idx = jnp.transpose(idx4[:, :, :HW], (2, 1, 0)).reshape(HW, TAPS4)
    wts = jnp.transpose(wts4[:, :, :HW], (2, 1, 0)).reshape(HW, TAPS4)
    return idx, wts


def kernel(x, W_off, b_off, W_d, b_d):
    x_flat = x[0].reshape(C, NFLAT)
    x_flat_p = jnp.pad(x_flat, ((0, 0), (0, 512)))
    shifts = [i * 224 + j for i in range(3) for j in range(3)]
    x9 = jnp.concatenate([x_flat_p[:, s:s + NFLAT] for s in shifts], axis=0)
    W9 = jnp.pad(jnp.transpose(W_off, (0, 2, 3, 1)).reshape(18, 9 * C),
                 ((0, 14), (0, 0)))
    b9 = jnp.pad(b_off, (0, 14)).reshape(32, 1)
    off_flat = _offset_conv(x9, W9, b9)
    offset = off_flat[:18].reshape(18, 224, 224)[:, :222, :222]

    idx, wts = _indices_and_weights(offset)
    idx = jnp.pad(idx, ((0, NPIX2 - HW), (0, 0))).reshape(-1)
    wts = jnp.pad(wts, ((0, NPIX2 - HW), (0, 0))).reshape(-1)

    x_pad = jnp.pad(x[0], ((0, 0), (PAD, PAD), (PAD, PAD)))
    x_padT = jnp.pad(x_pad.reshape(C, NROWS).T, ((0, RPAD - NROWS), (0, 0)))
    Wn = jnp.transpose(W_d.reshape(C, C, NTAP), (2, 1, 0))
    Wn = jnp.pad(Wn, ((0, 0), (0, 0), (0, 128 - C)))
    table = _table_matmul(x_padT, Wn).reshape(NTAP * RPAD, 128)

    out_rows = _sc_gather_combine(table, idx, wts)

    out = out_rows[:HW].T + b_d[:, None]
    return out.reshape(1, C, H, H)


# speedup vs baseline: 1.5963x; 1.2715x over previous
import functools

import jax
import jax.numpy as jnp
import numpy as np
from jax import lax
from jax.experimental import pallas as pl
from jax.experimental.pallas import tpu as pltpu
from jax.experimental.pallas import tpu_sc as plsc

KS = 3
PAD = 1
C = 96
NTAP = 9
H = 222
HW = H * H
HP = 226
NROWS = HP * HP
RPAD = 51200
NFLAT = 224 * 224

NW = 32
P_TILE = 1568
NPIX = NW * P_TILE
PB = 8
NB = P_TILE // PB
TAPS4 = 4 * NTAP
BLK = PB * TAPS4
GCH = 128
NG = (BLK + GCH - 1) // GCH
NPIX2 = NPIX + 4 * PB

_mesh = plsc.VectorSubcoreMesh(core_axis_name="c", subcore_axis_name="s")


@functools.partial(
    pl.kernel,
    mesh=_mesh,
    out_type=jax.ShapeDtypeStruct((NPIX, C), jnp.float32),
    scratch_types=[
        pltpu.VMEM((BLK,), jnp.int32),
        pltpu.VMEM((BLK,), jnp.int32),
        pltpu.VMEM((BLK + 16,), jnp.float32),
        pltpu.VMEM((BLK + 16,), jnp.float32),
        pltpu.VMEM((BLK, 128), jnp.float32),
        pltpu.VMEM((BLK, 128), jnp.float32),
        pltpu.VMEM((PB, C), jnp.float32),
        pltpu.SemaphoreType.DMA,
        pltpu.SemaphoreType.DMA,
        pltpu.SemaphoreType.DMA,
        pltpu.SemaphoreType.DMA,
    ],
)
def _sc_gather_combine(table_hbm, idx_hbm, w_hbm, out_hbm,
                       idx_v0, idx_v1, w_v0, w_v1, rows_v0, rows_v1, out_v,
                       sem_l0, sem_l1, sem_g0, sem_g1):
    wid = lax.axis_index("s") * 2 + lax.axis_index("c")
    base_pix = wid * P_TILE
    idx_v = (idx_v0, idx_v1)
    w_v = (w_v0, w_v1)
    rows_v = (rows_v0, rows_v1)
    sem_l = (sem_l0, sem_l1)
    sem_g = (sem_g0, sem_g1)

    def load_copies(b, buf):
        off = (base_pix + b * PB) * TAPS4
        return (pltpu.make_async_copy(idx_hbm.at[pl.ds(off, BLK)],
                                      idx_v[buf], sem_l[buf]),
                pltpu.make_async_copy(w_hbm.at[pl.ds(off, BLK)],
                                     w_v[buf].at[pl.ds(0, BLK)], sem_l[buf]))

    def gather_copies(buf):
        out = []
        for k in range(NG):
            cnt = min(GCH, BLK - k * GCH)
            out.append(pltpu.make_async_copy(
                table_hbm.at[idx_v[buf].at[pl.ds(k * GCH, cnt)]],
                rows_v[buf].at[pl.ds(k * GCH, cnt)], sem_g[buf]))
        return out

    for c0 in load_copies(0, 0):
        c0.start()
    for c0 in load_copies(1, 1):
        c0.start()
    for c0 in load_copies(0, 0):
        c0.wait()
    for c0 in gather_copies(0):
        c0.start()

    def combine(buf, b):
        def pix_body(i, c2):
            base = i * TAPS4
            wvecs = tuple(w_v[buf][pl.ds(base + 16 * g, 16)] for g in range(3))
            acc = [jnp.zeros((16,), jnp.float32) for _ in range(C // 16)]
            for t in range(TAPS4):
                wv = lax.broadcast_in_dim(wvecs[t // 16][t % 16], (16,), ())
                for c in range(C // 16):
                    acc[c] = acc[c] + wv * rows_v[buf][base + t,
                                                       pl.ds(c * 16, 16)]
            for c in range(C // 16):
                out_v[i, pl.ds(c * 16, 16)] = acc[c]
            return c2
        lax.fori_loop(0, PB, pix_body, 0)
        pltpu.sync_copy(out_v, out_hbm.at[pl.ds(base_pix + b * PB, PB)])

    def pair(i, carry):
        for phase in range(2):
            b = 2 * i + phase
            nbuf = 1 - phase
            for c0 in load_copies(b + 1, nbuf):
                c0.wait()
            for c0 in gather_copies(phase):
                c0.wait()
            for c0 in gather_copies(nbuf):
                c0.start()
            combine(phase, b)
            for c0 in load_copies(b + 2, phase):
                c0.start()
        return carry

    lax.fori_loop(0, NB // 2, pair, 0)

    for c0 in gather_copies(0):
        c0.wait()
    for c0 in load_copies(NB + 1, 1):
        c0.wait()


def _offset_conv_body(x9_ref, w_ref, b_ref, o_ref):
    o_ref[...] = jnp.dot(w_ref[...], x9_ref[...],
                         preferred_element_type=jnp.float32) + b_ref[...]


def _offset_conv(x9, W9, b9):
    return pl.pallas_call(
        _offset_conv_body,
        grid=(NFLAT // 512,),
        in_specs=[pl.BlockSpec((9 * C, 512), lambda j: (0, j)),
                  pl.BlockSpec((32, 9 * C), lambda j: (0, 0)),
                  pl.BlockSpec((32, 1), lambda j: (0, 0))],
        out_specs=pl.BlockSpec((32, 512), lambda j: (0, j)),
        out_shape=jax.ShapeDtypeStruct((32, NFLAT), jnp.float32),
    )(x9, W9, b9)


def _table_body(x_ref, w_ref, o_ref):
    o_ref[0] = jnp.dot(x_ref[...], w_ref[0],
                       preferred_element_type=jnp.float32)


def _table_matmul(x_padT, Wn):
    return pl.pallas_call(
        _table_body,
        grid=(RPAD // 512, NTAP),
        in_specs=[pl.BlockSpec((512, C), lambda r, n: (r, 0)),
                  pl.BlockSpec((1, C, 128), lambda r, n: (n, 0, 0))],
        out_specs=pl.BlockSpec((1, 512, 128), lambda r, n: (n, r, 0)),
        out_shape=jax.ShapeDtypeStruct((NTAP, RPAD, 128), jnp.float32),
    )(x_padT, Wn)


HWP = 49664


def _iw_body(ox_ref, oy_ref, dd_ref, hh_ref, ww_ref, idx_ref, wts_ref):
    dd = dd_ref[...]
    px = ox_ref[...] + dd[:, 0:1] + hh_ref[...]
    py = oy_ref[...] + dd[:, 1:2] + ww_ref[...]

    def axis_terms(p):
        f = jnp.floor(p)
        q0 = jnp.clip(f, 0.0, HP - 1.0)
        q1 = jnp.clip(f + 1.0, 0.0, HP - 1.0)
        masked = jnp.logical_or(p < float(PAD), p > HP - 1.0 - PAD)
        pu = jnp.clip(jnp.where(masked, f, p), 0.0, HP - 1.0)
        return q0, q1, 1.0 + q0 - pu, 1.0 - (q1 - pu)

    qx0, qx1, wx0, wx1 = axis_terms(px)
    qy0, qy1, wy0, wy1 = axis_terms(py)
    nbase = dd[:, 2:3]
    for t, (qx, qy, wx, wy) in enumerate(
            [(qx0, qy0, wx0, wy0), (qx1, qy1, wx1, wy1),
             (qx0, qy1, wx0, wy1), (qx1, qy0, wx1, wy0)]):
        idx_ref[t] = (nbase + qx * float(HP) + qy).astype(jnp.int32)
        wts_ref[t] = wx * wy


def _iw_kernel(ox, oy, dd, hh, ww):
    return pl.pallas_call(
        _iw_body,
        grid=(HWP // 512,),
        in_specs=[pl.BlockSpec((NTAP, 512), lambda j: (0, j)),
                  pl.BlockSpec((NTAP, 512), lambda j: (0, j)),
                  pl.BlockSpec((NTAP, 4), lambda j: (0, 0)),
                  pl.BlockSpec((1, 512), lambda j: (0, j)),
                  pl.BlockSpec((1, 512), lambda j: (0, j))],
        out_specs=[pl.BlockSpec((4, NTAP, 512), lambda j: (0, 0, j)),
                   pl.BlockSpec((4, NTAP, 512), lambda j: (0, 0, j))],
        out_shape=[jax.ShapeDtypeStruct((4, NTAP, HWP), jnp.int32),
                   jax.ShapeDtypeStruct((4, NTAP, HWP), jnp.float32)],
    )(ox, oy, dd, hh, ww)


def _indices_and_weights(offset):
    ox = jnp.pad(offset[0::2].reshape(NTAP, HW), ((0, 0), (0, HWP - HW)))
    oy = jnp.pad(offset[1::2].reshape(NTAP, HW), ((0, 0), (0, HWP - HW)))
    j = np.arange(HWP)
    hh = jnp.asarray((np.minimum(j // H, H - 1) + 1).reshape(1, -1),
                     jnp.float32)
    ww = jnp.asarray((j % H + 1).reshape(1, -1), jnp.float32)
    dd = np.zeros((NTAP, 4), np.float32)
    dd[:, 0] = np.repeat(np.arange(-1, 2), 3)
    dd[:, 1] = np.tile(np.arange(-1, 2), 3)
    dd[:, 2] = np.arange(NTAP) * float(RPAD)
    idx4, wts4 = _iw_kernel(ox, oy, jnp.asarray(dd), hh, ww)
    idx = jnp.transpose(idx4[:, :, :HW], (2, 1, 0)).reshape(HW, TAPS4)
    wts = jnp.transpose(wts4[:, :, :HW], (2, 1, 0)).reshape(HW, TAPS4)
    return idx, wts


def kernel(x, W_off, b_off, W_d, b_d):
    x_flat = x[0].reshape(C, NFLAT)
    x_flat_p = jnp.pad(x_flat, ((0, 0), (0, 512)))
    shifts = [i * 224 + j for i in range(3) for j in range(3)]
    x9 = jnp.concatenate([x_flat_p[:, s:s + NFLAT] for s in shifts], axis=0)
    W9 = jnp.pad(jnp.transpose(W_off, (0, 2, 3, 1)).reshape(18, 9 * C),
                 ((0, 14), (0, 0)))
    b9 = jnp.pad(b_off, (0, 14)).reshape(32, 1)
    off_flat = _offset_conv(x9, W9, b9)
    offset = off_flat[:18].reshape(18, 224, 224)[:, :222, :222]

    idx, wts = _indices_and_weights(offset)
    idx = jnp.pad(idx, ((0, NPIX2 - HW), (0, 0))).reshape(-1)
    wts = jnp.pad(wts, ((0, NPIX2 - HW), (0, 0))).reshape(-1)

    x_pad = jnp.pad(x[0], ((0, 0), (PAD, PAD), (PAD, PAD)))
    x_padT = jnp.pad(x_pad.reshape(C, NROWS).T, ((0, RPAD - NROWS), (0, 0)))
    Wn = jnp.transpose(W_d.reshape(C, C, NTAP), (2, 1, 0))
    Wn = jnp.pad(Wn, ((0, 0), (0, 0), (0, 128 - C)))
    table = _table_matmul(x_padT, Wn).reshape(NTAP * RPAD, 128)

    out_rows = _sc_gather_combine(table, idx, wts)

    out = out_rows[:HW].T + b_d[:, None]
    return out.reshape(1, C, H, H)
